# Initial kernel scaffold; baseline (speedup 1.0000x reference)
import jax
import jax.numpy as jnp
from jax.experimental import pallas as pl


def kernel(X, train_features, train_labels):
    raise NotImplementedError("write your pallas kernel here")


# speedup vs baseline: 5.3944x; 5.3944x over previous
import functools

import jax
import jax.numpy as jnp
from jax import lax
from jax.experimental import pallas as pl

Q = 128
D = 64
K = 16384
NUM_LABELS = 1000
TOP_K = 16

_INTERPRET = False


def _knn_body(x_ref, t_ref, lab_ref, out_ref):
    x = x_ref[...]
    t = t_ref[...]
    labels = lab_ref[...]

    t2 = jnp.sum(t * t, axis=0, keepdims=True)
    xt = lax.dot_general(
        x, t, (((1,), (0,)), ((), ())),
        preferred_element_type=jnp.float32,
        precision=lax.Precision.HIGHEST,
    )
    s = t2 - 2.0 * xt

    iota = lax.broadcasted_iota(jnp.int32, (Q, K), 1)
    liota = lax.broadcasted_iota(jnp.int32, (Q, NUM_LABELS), 1)
    votes = jnp.zeros((Q, NUM_LABELS), dtype=jnp.float32)

    for _ in range(TOP_K):
        minv = jnp.min(s, axis=1, keepdims=True)
        eq = s == minv
        idx = jnp.min(jnp.where(eq, iota, K), axis=1, keepdims=True)
        sel = iota == idx
        lab = jnp.max(jnp.where(sel, labels, 0), axis=1, keepdims=True)
        votes = votes + (lab == liota).astype(jnp.float32)
        s = jnp.where(sel, jnp.inf, s)

    vmax = jnp.max(votes, axis=1, keepdims=True)
    out = jnp.min(jnp.where(votes == vmax, liota, NUM_LABELS),
                  axis=1, keepdims=True)
    out_ref[...] = out


@jax.jit
def kernel(X, train_features, train_labels):
    t = train_features[0]
    labels = train_labels.reshape(1, K)
    out = pl.pallas_call(
        _knn_body,
        out_shape=jax.ShapeDtypeStruct((Q, 1), jnp.int32),
        interpret=_INTERPRET,
    )(X.astype(jnp.float32), t, labels)
    return out[:, 0]


# speedup vs baseline: 7.6450x; 1.4172x over previous
import jax
import jax.numpy as jnp
from jax import lax
from jax.experimental import pallas as pl

Q = 128
D = 64
K = 16384
NUM_LABELS = 1000
TOP_K = 16
B = 128
W = K // B

_INTERPRET = False


def _rmin2(x):
    return jnp.min(jnp.min(x, axis=2, keepdims=True), axis=1, keepdims=True)


def _rmax2(x):
    return jnp.max(jnp.max(x, axis=2, keepdims=True), axis=1, keepdims=True)


def _knn_body(x_ref, t_ref, lab_ref, out_ref):
    x = x_ref[...]
    t = t_ref[...]
    labf = lab_ref[...]

    t2 = jnp.sum(t * t, axis=0, keepdims=True)
    xt = lax.dot_general(
        x, t, (((1,), (0,)), ((), ())),
        preferred_element_type=jnp.float32,
        precision=lax.Precision.HIGHEST,
    )
    s = t2 - 2.0 * xt

    s3 = s.reshape(Q, B, W)

    m = jnp.min(s3, axis=2)
    biota = lax.broadcasted_iota(jnp.int32, (Q, B), 1)
    bids = []
    for _ in range(TOP_K):
        mn = jnp.min(m, axis=1, keepdims=True)
        beq = m == mn
        bid = jnp.min(jnp.where(beq, biota, B), axis=1, keepdims=True)
        bids.append(bid)
        m = jnp.where(biota == bid, jnp.inf, m)
    bids = jnp.concatenate(bids, axis=1)

    oh = (bids[:, :, None] ==
          lax.broadcasted_iota(jnp.int32, (Q, TOP_K, B), 2)).astype(jnp.float32)
    g = lax.dot_general(
        oh, s3, (((2,), (1,)), ((0,), (0,))),
        preferred_element_type=jnp.float32,
        precision=lax.Precision.HIGHEST,
    )
    glab = lax.dot_general(
        oh.reshape(Q * TOP_K, B), labf, (((1,), (0,)), ((), ())),
        preferred_element_type=jnp.float32,
        precision=lax.Precision.HIGHEST,
    ).reshape(Q, TOP_K, W)

    cidx = (bids[:, :, None] * W +
            lax.broadcasted_iota(jnp.int32, (Q, TOP_K, W), 2))

    liota = lax.broadcasted_iota(jnp.int32, (Q, NUM_LABELS), 1)
    votes = jnp.zeros((Q, NUM_LABELS), dtype=jnp.float32)
    for _ in range(TOP_K):
        mn = _rmin2(g)
        eq = g == mn
        gi = jnp.min(jnp.min(jnp.where(eq, cidx, K), axis=2, keepdims=True),
                     axis=1, keepdims=True)
        sel = cidx == gi
        lab = _rmax2(jnp.where(sel, glab, 0.0))
        votes = votes + (lab[:, :, 0] == liota.astype(jnp.float32)).astype(jnp.float32)
        g = jnp.where(sel, jnp.inf, g)

    vmax = jnp.max(votes, axis=1, keepdims=True)
    out = jnp.min(jnp.where(votes == vmax, liota, NUM_LABELS),
                  axis=1, keepdims=True)
    out_ref[...] = out


@jax.jit
def kernel(X, train_features, train_labels):
    t = train_features[0]
    labf = train_labels.reshape(B, W).astype(jnp.float32)
    out = pl.pallas_call(
        _knn_body,
        out_shape=jax.ShapeDtypeStruct((Q, 1), jnp.int32),
        interpret=_INTERPRET,
    )(X.astype(jnp.float32), t, labf)
    return out[:, 0]
